# baseline (speedup 1.0000x reference)
import jax
import jax.numpy as jnp
from jax.experimental import pallas as pl

_GEN_SIZES = (864, 18432, 36864, 32, 64, 64, 36864, 18432, 864, 64, 32, 3)
_TOTAL = sum(_GEN_SIZES)


def _prologue_body(x_ref, we_ref, be_ref, cb_ref, cbt_ref, w1_ref, b1_ref,
                   w2_ref, b2_ref, h2_ref, loss_ref):
    x = x_ref[...]
    embed = jnp.dot(x, we_ref[...], preferred_element_type=jnp.float32) + be_ref[...]
    cb = cb_ref[...]
    cbt = cbt_ref[...]
    scores = jnp.sum(cbt * cbt, axis=0)[None, :] - 2.0 * jnp.dot(
        embed, cbt, preferred_element_type=jnp.float32)
    idx = jnp.argmin(scores, axis=1)
    onehot = (jax.lax.broadcasted_iota(jnp.int32, scores.shape, 1)
              == idx[:, None]).astype(jnp.float32)
    quantized = jnp.dot(onehot, cb, preferred_element_type=jnp.float32)
    diff = quantized - embed
    loss = 1.25 * jnp.mean(diff * diff)
    loss_ref[...] = loss[None, None]
    h1 = jnp.dot(quantized, w1_ref[...], preferred_element_type=jnp.float32) + b1_ref[...]
    h1 = jnp.where(h1 >= 0, h1, 0.01 * h1)
    h2 = jnp.dot(h1, w2_ref[...], preferred_element_type=jnp.float32) + b2_ref[...]
    h2_ref[...] = jnp.where(h2 >= 0, h2, 0.01 * h2)


def _gen_body(h_ref, w_ref, out_ref):
    out_ref[...] = jnp.dot(h_ref[...], w_ref[...],
                           preferred_element_type=jnp.float32)


def kernel(input, W_enc, b_enc, codebook, W_h1, b_h1, W_h2, b_h2,
           ek0, ek1, ek2, eb0, eb1, eb2, dk0, dk1, dk2, db0, db1, db2):
    B, EMB = input.shape[0], W_enc.shape[1]
    h2, loss = pl.pallas_call(
        _prologue_body,
        out_shape=(
            jax.ShapeDtypeStruct((B, EMB), jnp.float32),
            jax.ShapeDtypeStruct((1, 1), jnp.float32),
        ),
    )(input, W_enc, b_enc.reshape(1, EMB), codebook, codebook.T,
      W_h1, b_h1.reshape(1, -1), W_h2, b_h2.reshape(1, EMB))

    W_cat = jnp.concatenate(
        [ek0, ek1, ek2, eb0, eb1, eb2, dk0, dk1, dk2, db0, db1, db2], axis=1)

    TN = 2048
    NT = (_TOTAL + TN - 1) // TN
    out = pl.pallas_call(
        _gen_body,
        grid=(NT,),
        in_specs=[
            pl.BlockSpec((B, EMB), lambda i: (0, 0)),
            pl.BlockSpec((EMB, TN), lambda i: (0, i)),
        ],
        out_specs=pl.BlockSpec((B, TN), lambda i: (0, i)),
        out_shape=jax.ShapeDtypeStruct((B, _TOTAL), jnp.float32),
    )(h2, W_cat)

    return (loss.reshape(()), out)


# speedup vs baseline: 2.4889x; 2.4889x over previous
import jax
import jax.numpy as jnp
from jax import lax
from jax.experimental import pallas as pl
from jax.experimental.pallas import tpu as pltpu

_TOTAL = 112579
_TN = 2048
_NSTEPS = 9 + 18 + 18 + 9
_SHIFT = 96
_HEAD = _TN - _SHIFT

_R_A, _W_A = 0, 896
_R_C, _W_C = 56064, 256
_R_D, _W_D = 111616, 963
_W_REPAIR = _W_A + _W_C + _W_D


def _prologue_body(x_ref, we_ref, be_ref, cb_ref, cbt_ref, w1_ref, b1_ref,
                   w2_ref, b2_ref, h2_ref, loss_ref):
    x = x_ref[...]
    embed = jnp.dot(x, we_ref[...], preferred_element_type=jnp.float32) + be_ref[...]
    cb = cb_ref[...]
    cbt = cbt_ref[...]
    scores = jnp.sum(cbt * cbt, axis=0)[None, :] - 2.0 * jnp.dot(
        embed, cbt, preferred_element_type=jnp.float32)
    idx = jnp.argmin(scores, axis=1)
    onehot = (jax.lax.broadcasted_iota(jnp.int32, scores.shape, 1)
              == idx[:, None]).astype(jnp.float32)
    quantized = jnp.dot(onehot, cb, preferred_element_type=jnp.float32)
    diff = quantized - embed
    loss = 1.25 * jnp.mean(diff * diff)
    loss_ref[...] = loss[None, None]
    h1 = jnp.dot(quantized, w1_ref[...], preferred_element_type=jnp.float32) + b1_ref[...]
    h1 = jnp.where(h1 >= 0, h1, 0.01 * h1)
    h2 = jnp.dot(h1, w2_ref[...], preferred_element_type=jnp.float32) + b2_ref[...]
    h2_ref[...] = jnp.where(h2 >= 0, h2, 0.01 * h2)


def _big_body(h_ref, w1_ref, w2_ref, w3_ref, w4_ref, out_ref, mm, dbuf, sem):
    i = pl.program_id(0)
    s2 = lax.rem(i, 2)
    s3 = lax.rem(i, 3)
    p3 = lax.rem(i + 2, 3)

    @pl.when(i >= 2)
    def _wait_prev():
        pltpu.make_async_copy(dbuf.at[s2], out_ref.at[:, pl.ds(0, _TN)],
                              sem.at[s2]).wait()

    h = h_ref[...]

    @pl.when(i < 9)
    def _s1():
        mm[s3] = jnp.dot(h, w1_ref[...], preferred_element_type=jnp.float32)

    @pl.when((i >= 9) & (i < 27))
    def _s2():
        mm[s3] = jnp.dot(h, w2_ref[...], preferred_element_type=jnp.float32)

    @pl.when((i >= 27) & (i < 45))
    def _s3():
        mm[s3] = jnp.dot(h, w3_ref[...], preferred_element_type=jnp.float32)

    @pl.when(i >= 45)
    def _s4():
        mm[s3] = jnp.dot(h, w4_ref[...], preferred_element_type=jnp.float32)

    col = pl.multiple_of(_TN * i + jnp.where(i < 27, 768, 1024), 128)

    @pl.when(i < 27)
    def _shifted():
        dbuf[s2] = jnp.concatenate(
            [mm[p3, :, _HEAD:], mm[s3, :, :_HEAD]], axis=1)
        pltpu.make_async_copy(dbuf.at[s2], out_ref.at[:, pl.ds(col, _TN)],
                              sem.at[s2]).start()

    @pl.when(i >= 27)
    def _aligned():
        pltpu.make_async_copy(mm.at[s3], out_ref.at[:, pl.ds(col, _TN)],
                              sem.at[s2]).start()

    @pl.when(i == _NSTEPS - 1)
    def _drain():
        pltpu.make_async_copy(dbuf.at[1 - s2], out_ref.at[:, pl.ds(0, _TN)],
                              sem.at[1 - s2]).wait()
        pltpu.make_async_copy(dbuf.at[s2], out_ref.at[:, pl.ds(0, _TN)],
                              sem.at[s2]).wait()


def _repair_body(o_in_ref, h_ref, wr_ref, out_ref, rbuf, sem):
    rbuf[...] = jnp.dot(h_ref[...], wr_ref[...],
                        preferred_element_type=jnp.float32)
    c1 = pltpu.make_async_copy(rbuf.at[:, pl.ds(0, _W_A)],
                               out_ref.at[:, pl.ds(_R_A, _W_A)], sem.at[0])
    c2 = pltpu.make_async_copy(rbuf.at[:, pl.ds(_W_A, _W_C)],
                               out_ref.at[:, pl.ds(_R_C, _W_C)], sem.at[1])
    c3 = pltpu.make_async_copy(rbuf.at[:, pl.ds(_W_A + _W_C, _W_D)],
                               out_ref.at[:, pl.ds(_R_D, _W_D)], sem.at[2])
    c1.start(); c2.start(); c3.start()
    c1.wait(); c2.wait(); c3.wait()


def kernel(input, W_enc, b_enc, codebook, W_h1, b_h1, W_h2, b_h2,
           ek0, ek1, ek2, eb0, eb1, eb2, dk0, dk1, dk2, db0, db1, db2):
    B, EMB = input.shape[0], W_enc.shape[1]
    h2, loss = pl.pallas_call(
        _prologue_body,
        out_shape=(
            jax.ShapeDtypeStruct((B, EMB), jnp.float32),
            jax.ShapeDtypeStruct((1, 1), jnp.float32),
        ),
    )(input, W_enc, b_enc.reshape(1, EMB), codebook, codebook.T,
      W_h1, b_h1.reshape(1, -1), W_h2, b_h2.reshape(1, EMB))

    big = pl.pallas_call(
        _big_body,
        grid=(_NSTEPS,),
        in_specs=[
            pl.BlockSpec((B, EMB), lambda i: (0, 0)),
            pl.BlockSpec((EMB, _TN), lambda i: (0, jnp.clip(i, 0, 8))),
            pl.BlockSpec((EMB, _TN), lambda i: (0, jnp.clip(i - 9, 0, 17))),
            pl.BlockSpec((EMB, _TN), lambda i: (0, jnp.clip(i - 27, 0, 17))),
            pl.BlockSpec((EMB, _TN), lambda i: (0, jnp.clip(i - 45, 0, 8))),
        ],
        out_specs=pl.BlockSpec(memory_space=pl.ANY),
        out_shape=jax.ShapeDtypeStruct((B, _TOTAL), jnp.float32),
        scratch_shapes=[
            pltpu.VMEM((3, B, _TN), jnp.float32),
            pltpu.VMEM((2, B, _TN), jnp.float32),
            pltpu.SemaphoreType.DMA((2,)),
        ],
    )(h2, ek1, ek2, dk0, dk1)

    W_repair = jnp.concatenate(
        [ek0, ek1[:, :32], ek2[:, 36768:], eb0, eb1, eb2,
         dk2, db0, db1, db2], axis=1)

    out = pl.pallas_call(
        _repair_body,
        in_specs=[
            pl.BlockSpec(memory_space=pl.ANY),
            pl.BlockSpec((B, EMB), lambda: (0, 0)),
            pl.BlockSpec((EMB, _W_REPAIR), lambda: (0, 0)),
        ],
        out_specs=pl.BlockSpec(memory_space=pl.ANY),
        out_shape=jax.ShapeDtypeStruct((B, _TOTAL), jnp.float32),
        input_output_aliases={0: 0},
        scratch_shapes=[
            pltpu.VMEM((B, _W_REPAIR), jnp.float32),
            pltpu.SemaphoreType.DMA((3,)),
        ],
    )(big, h2, W_repair)

    return (loss.reshape(()), out)
